# baseline (speedup 1.0000x reference)
import functools

import numpy as np
import jax
import jax.numpy as jnp
from jax import lax
from jax.experimental import pallas as pl
from jax.experimental.pallas import tpu as pltpu
from jax.experimental.pallas import tpu_sc as plsc

_N_WIDTH = 256
_N_ORDER = 8
_N_ELEMENTS = 64
_N_NODES = _N_ELEMENTS * _N_ORDER + 1
_N_COLL = 100
_DELTA_X = 0.5 * _N_ORDER / (_N_NODES - 1)
_NODES = np.linspace(-1.0, 1.0, _N_ORDER + 1)
_NB = _N_ORDER + 1

_N_WORKERS = 32
_HALF = 128
_N_ITEMS = 3 * _N_COLL * 2
_ITEMS_PER_W = 19


def _inv_table():
    v = np.zeros((16, _NB), np.float32)
    for j in range(_NB):
        for m in range(_NB):
            if j != m:
                v[j, m] = 1.0 / (_NODES[j] - _NODES[m])
    return v


_INV_TABLE = _inv_table()


def _omit_one_products(fs):
    n = len(fs)
    pre = [None] * n
    suf = [None] * n
    acc = None
    for i in range(n):
        pre[i] = acc
        acc = fs[i] if acc is None else acc * fs[i]
    acc = None
    for i in range(n - 1, -1, -1):
        suf[i] = acc
        acc = fs[i] if acc is None else acc * fs[i]
    out = []
    for i in range(n):
        if pre[i] is None:
            out.append(suf[i])
        elif suf[i] is None:
            out.append(pre[i])
        else:
            out.append(pre[i] * suf[i])
    return out


def _basis_rows(x_t, inv_tab):
    j2 = jax.lax.broadcasted_iota(jnp.int32, (16, _N_WIDTH), 0)
    xb = jnp.broadcast_to(x_t, (16, _N_WIDTH)).astype(jnp.float32)
    f = []
    cs = [inv_tab[:, m : m + 1] for m in range(_NB)]
    for m in range(_NB):
        f.append(jnp.where(j2 == m, 1.0, (xb - np.float32(_NODES[m])) * cs[m]))

    phi = f[0]
    for m in range(1, _NB):
        phi = phi * f[m]

    p1 = _omit_one_products(f)
    dphi = cs[0] * p1[0]
    for i in range(1, _NB):
        dphi = dphi + cs[i] * p1[i]

    ddphi = None
    for i in range(_NB):
        idxs = [m for m in range(_NB) if m != i]
        q = _omit_one_products([f[m] for m in idxs])
        inner = None
        for pos, m in enumerate(idxs):
            term = cs[m] * q[pos]
            inner = term if inner is None else inner + term
        term = cs[i] * inner
        ddphi = term if ddphi is None else ddphi + term

    dphi = dphi * np.float32(1.0 / _DELTA_X)
    ddphi = ddphi * np.float32(1.0 / (_DELTA_X * _DELTA_X))
    return phi, dphi, ddphi


def _sc_fill_body(phi_o, dphi_o, ddphi_o, dummy_o, zbuf, sem):
    wid = lax.axis_index("s") * 2 + lax.axis_index("c")

    def _zrow(r, carry):
        for c in range(32):
            zbuf[0, r, pl.ds(c * 16, 16)] = jnp.zeros((16,), jnp.float32)
        zbuf[0, r, pl.ds(_N_NODES - 16, 16)] = jnp.zeros((16,), jnp.float32)
        return carry

    lax.fori_loop(0, _HALF, _zrow, 0)

    outs = (phi_o, dphi_o, ddphi_o)
    for n in range(_ITEMS_PER_W):
        h = wid * _ITEMS_PER_W + n
        b = h // (2 * _N_COLL)
        rem = h % (2 * _N_COLL)
        row = rem // 2
        half = rem % 2
        for b_id in range(3):
            cond = (h < _N_ITEMS) & (b == b_id)
            dst = outs[b_id].at[
                pl.ds(row, 1), pl.ds(half * _HALF, _HALF), pl.ds(0, _N_NODES)
            ]

            @pl.when(cond)
            def _(dst=dst):
                pltpu.make_async_copy(zbuf, dst, sem).start()

        @pl.when(h >= _N_ITEMS)
        def _():
            pltpu.make_async_copy(zbuf, dummy_o, sem).start()

    for n in range(_ITEMS_PER_W):
        pltpu.make_async_copy(zbuf, dummy_o, sem).wait()


_big = jax.ShapeDtypeStruct((_N_COLL, _N_WIDTH, _N_NODES), jnp.float32)
_vec = jax.ShapeDtypeStruct((1, _N_WIDTH), jnp.float32)

_sc_fill_call = functools.partial(
    pl.kernel,
    mesh=plsc.VectorSubcoreMesh(
        core_axis_name="c", subcore_axis_name="s", num_cores=2, num_subcores=16
    ),
    out_type=[
        _big,
        _big,
        _big,
        jax.ShapeDtypeStruct((1, _HALF, _N_NODES), jnp.float32),
    ],
    scratch_types=[
        pltpu.VMEM((1, _HALF, _N_NODES), jnp.float32),
        pltpu.SemaphoreType.DMA,
    ],
)(_sc_fill_body)


def _insert_body(
    s_ref,
    x_ref,
    w_ref,
    inv_ref,
    phi_in,
    dphi_in,
    ddphi_in,
    t_r,
    dt_r,
    ddt_r,
    phi_o,
    dphi_o,
    ddphi_o,
    rb0,
    rb1,
    rb2,
    sem_r,
):
    del phi_in, dphi_in, ddphi_in
    xs = x_ref[...]
    x_shift = (_N_NODES - 1.0) * xs
    id_el = jnp.clip(jnp.floor(x_shift / _N_ORDER), 0.0, _N_ELEMENTS - 1.0)
    nl_f = id_el * _N_ORDER
    x_t = (x_shift - nl_f - 0.5 * _N_ORDER) / (0.5 * _N_ORDER)
    phi, dphi, ddphi = _basis_rows(x_t, inv_ref[...])

    ident = (
        jax.lax.broadcasted_iota(jnp.int32, (_N_WIDTH, _N_WIDTH), 0)
        == jax.lax.broadcasted_iota(jnp.int32, (_N_WIDTH, _N_WIDTH), 1)
    ).astype(jnp.float32)

    def to_col(row):
        return jnp.sum(ident * row, axis=1, keepdims=True)

    def to_row(col):
        return jnp.sum(ident * col, axis=0, keepdims=True)

    nl_col = to_col(nl_f).astype(jnp.int32)
    rel = jax.lax.broadcasted_iota(jnp.int32, (_N_WIDTH, _N_NODES), 1) - nl_col

    bases = (phi, dphi, ddphi)
    dense = [jnp.zeros((_N_WIDTH, _N_NODES), jnp.float32) for _ in range(3)]
    for j in range(_NB):
        mask = rel == j
        for b in range(3):
            colv = to_col(bases[b][j : j + 1, :])
            dense[b] = jnp.where(mask, colv, dense[b])

    rbs = (rb0, rb1, rb2)
    for rb, d in zip(rbs, dense):
        rb[...] = d[None]

    s = s_ref[0]
    outs = (phi_o, dphi_o, ddphi_o)
    for rb, o in zip(rbs, outs):
        pltpu.make_async_copy(rb, o.at[pl.ds(s, 1)], sem_r).start()

    w = w_ref[...]
    for d, tr in zip(dense, (t_r, dt_r, ddt_r)):
        tcol = jnp.sum(w * d, axis=1, keepdims=True)
        tr[...] = to_row(tcol)

    for rb, o in zip(rbs, outs):
        pltpu.make_async_copy(rb, o.at[pl.ds(s, 1)], sem_r).wait()


_insert_call = pl.pallas_call(
    _insert_body,
    in_specs=[
        pl.BlockSpec(memory_space=pltpu.SMEM),
        pl.BlockSpec((1, _N_WIDTH), lambda: (0, 0)),
        pl.BlockSpec((_N_WIDTH, _N_NODES), lambda: (0, 0)),
        pl.BlockSpec((16, _NB), lambda: (0, 0)),
        pl.BlockSpec(memory_space=pltpu.MemorySpace.HBM),
        pl.BlockSpec(memory_space=pltpu.MemorySpace.HBM),
        pl.BlockSpec(memory_space=pltpu.MemorySpace.HBM),
    ],
    out_specs=[
        pl.BlockSpec((1, _N_WIDTH), lambda: (0, 0)),
        pl.BlockSpec((1, _N_WIDTH), lambda: (0, 0)),
        pl.BlockSpec((1, _N_WIDTH), lambda: (0, 0)),
        pl.BlockSpec(memory_space=pltpu.MemorySpace.HBM),
        pl.BlockSpec(memory_space=pltpu.MemorySpace.HBM),
        pl.BlockSpec(memory_space=pltpu.MemorySpace.HBM),
    ],
    out_shape=[_vec, _vec, _vec, _big, _big, _big],
    scratch_shapes=[
        pltpu.VMEM((1, _N_WIDTH, _N_NODES), jnp.float32),
        pltpu.VMEM((1, _N_WIDTH, _N_NODES), jnp.float32),
        pltpu.VMEM((1, _N_WIDTH, _N_NODES), jnp.float32),
        pltpu.SemaphoreType.DMA,
    ],
    input_output_aliases={4: 3, 5: 4, 6: 5},
)


def kernel(x, epoch, sample, weight):
    del epoch
    s = jnp.asarray(sample, jnp.int32).reshape((1,))
    zphi, zdphi, zddphi, _ = _sc_fill_call()
    t, dt, ddt, phi_buf, dphi_buf, ddphi_buf = _insert_call(
        s, x, weight, jnp.asarray(_INV_TABLE), zphi, zdphi, zddphi
    )
    return (t, dt, ddt, phi_buf, dphi_buf, ddphi_buf, jnp.float32(_DELTA_X))


# speedup vs baseline: 1.0265x; 1.0265x over previous
import functools

import numpy as np
import jax
import jax.numpy as jnp
from jax import lax
from jax.experimental import pallas as pl
from jax.experimental.pallas import tpu as pltpu
from jax.experimental.pallas import tpu_sc as plsc

_N_WIDTH = 256
_N_ORDER = 8
_N_ELEMENTS = 64
_N_NODES = _N_ELEMENTS * _N_ORDER + 1
_N_COLL = 100
_DELTA_X = 0.5 * _N_ORDER / (_N_NODES - 1)
_NODES = np.linspace(-1.0, 1.0, _N_ORDER + 1)
_NB = _N_ORDER + 1

_N_WORKERS = 32
_HALF = 128
_N_ITEMS = 2 * _N_COLL * 2
_ITEMS_PER_W = 13
_TC_BLK = 10


def _inv_table():
    v = np.zeros((16, _NB), np.float32)
    for j in range(_NB):
        for m in range(_NB):
            if j != m:
                v[j, m] = 1.0 / (_NODES[j] - _NODES[m])
    return v


_INV_TABLE = _inv_table()


def _omit_one_products(fs):
    n = len(fs)
    pre = [None] * n
    suf = [None] * n
    acc = None
    for i in range(n):
        pre[i] = acc
        acc = fs[i] if acc is None else acc * fs[i]
    acc = None
    for i in range(n - 1, -1, -1):
        suf[i] = acc
        acc = fs[i] if acc is None else acc * fs[i]
    out = []
    for i in range(n):
        if pre[i] is None:
            out.append(suf[i])
        elif suf[i] is None:
            out.append(pre[i])
        else:
            out.append(pre[i] * suf[i])
    return out


def _basis_rows(x_t, inv_tab):
    j2 = jax.lax.broadcasted_iota(jnp.int32, (16, _N_WIDTH), 0)
    xb = jnp.broadcast_to(x_t, (16, _N_WIDTH)).astype(jnp.float32)
    f = []
    cs = [inv_tab[:, m : m + 1] for m in range(_NB)]
    for m in range(_NB):
        f.append(jnp.where(j2 == m, 1.0, (xb - np.float32(_NODES[m])) * cs[m]))

    phi = f[0]
    for m in range(1, _NB):
        phi = phi * f[m]

    p1 = _omit_one_products(f)
    dphi = cs[0] * p1[0]
    for i in range(1, _NB):
        dphi = dphi + cs[i] * p1[i]

    ddphi = None
    for i in range(_NB):
        idxs = [m for m in range(_NB) if m != i]
        q = _omit_one_products([f[m] for m in idxs])
        inner = None
        for pos, m in enumerate(idxs):
            term = cs[m] * q[pos]
            inner = term if inner is None else inner + term
        term = cs[i] * inner
        ddphi = term if ddphi is None else ddphi + term

    dphi = dphi * np.float32(1.0 / _DELTA_X)
    ddphi = ddphi * np.float32(1.0 / (_DELTA_X * _DELTA_X))
    return phi, dphi, ddphi


def _sc_fill_body(dphi_o, ddphi_o, dummy_o, zbuf, sem):
    wid = lax.axis_index("s") * 2 + lax.axis_index("c")

    def _zrow(r, carry):
        for c in range(32):
            zbuf[0, r, pl.ds(c * 16, 16)] = jnp.zeros((16,), jnp.float32)
        zbuf[0, r, pl.ds(_N_NODES - 16, 16)] = jnp.zeros((16,), jnp.float32)
        return carry

    lax.fori_loop(0, _HALF, _zrow, 0)

    outs = (dphi_o, ddphi_o)
    for n in range(_ITEMS_PER_W):
        h = wid * _ITEMS_PER_W + n
        b = h // (2 * _N_COLL)
        rem = h % (2 * _N_COLL)
        row = rem // 2
        half = rem % 2
        for b_id in range(2):
            cond = (h < _N_ITEMS) & (b == b_id)
            dst = outs[b_id].at[
                pl.ds(row, 1), pl.ds(half * _HALF, _HALF), pl.ds(0, _N_NODES)
            ]

            @pl.when(cond)
            def _(dst=dst):
                pltpu.make_async_copy(zbuf, dst, sem).start()

        @pl.when(h >= _N_ITEMS)
        def _():
            pltpu.make_async_copy(zbuf, dummy_o, sem).start()

    for n in range(_ITEMS_PER_W):
        pltpu.make_async_copy(zbuf, dummy_o, sem).wait()


_big = jax.ShapeDtypeStruct((_N_COLL, _N_WIDTH, _N_NODES), jnp.float32)
_vec = jax.ShapeDtypeStruct((1, _N_WIDTH), jnp.float32)

_sc_fill_call = functools.partial(
    pl.kernel,
    mesh=plsc.VectorSubcoreMesh(
        core_axis_name="c", subcore_axis_name="s", num_cores=2, num_subcores=16
    ),
    out_type=[
        _big,
        _big,
        jax.ShapeDtypeStruct((1, _HALF, _N_NODES), jnp.float32),
    ],
    scratch_types=[
        pltpu.VMEM((1, _HALF, _N_NODES), jnp.float32),
        pltpu.SemaphoreType.DMA,
    ],
)(_sc_fill_body)


def _tc_main_body(
    s_ref,
    x_ref,
    w_ref,
    inv_ref,
    t_r,
    dt_r,
    ddt_r,
    phi_o,
    rb1_o,
    rb2_o,
    zblk,
    rb0,
    sem_z,
    sem_r,
):
    zblk[...] = jnp.zeros((_TC_BLK, _N_WIDTH, _N_NODES), jnp.float32)
    n_blocks = _N_COLL // _TC_BLK
    for i in range(n_blocks):
        pltpu.make_async_copy(
            zblk, phi_o.at[pl.ds(i * _TC_BLK, _TC_BLK)], sem_z
        ).start()

    xs = x_ref[...]
    x_shift = (_N_NODES - 1.0) * xs
    id_el = jnp.clip(jnp.floor(x_shift / _N_ORDER), 0.0, _N_ELEMENTS - 1.0)
    nl_f = id_el * _N_ORDER
    x_t = (x_shift - nl_f - 0.5 * _N_ORDER) / (0.5 * _N_ORDER)
    phi, dphi, ddphi = _basis_rows(x_t, inv_ref[...])

    ident = (
        jax.lax.broadcasted_iota(jnp.int32, (_N_WIDTH, _N_WIDTH), 0)
        == jax.lax.broadcasted_iota(jnp.int32, (_N_WIDTH, _N_WIDTH), 1)
    ).astype(jnp.float32)

    def to_col(row):
        return jnp.sum(ident * row, axis=1, keepdims=True)

    def to_row(col):
        return jnp.sum(ident * col, axis=0, keepdims=True)

    nl_col = to_col(nl_f).astype(jnp.int32)
    rel = jax.lax.broadcasted_iota(jnp.int32, (_N_WIDTH, _N_NODES), 1) - nl_col

    bases = (phi, dphi, ddphi)
    dense = [jnp.zeros((_N_WIDTH, _N_NODES), jnp.float32) for _ in range(3)]
    for j in range(_NB):
        mask = rel == j
        for b in range(3):
            colv = to_col(bases[b][j : j + 1, :])
            dense[b] = jnp.where(mask, colv, dense[b])

    rb0[...] = dense[0][None]
    rb1_o[...] = dense[1][None]
    rb2_o[...] = dense[2][None]

    w = w_ref[...]
    for d, tr in zip(dense, (t_r, dt_r, ddt_r)):
        tcol = jnp.sum(w * d, axis=1, keepdims=True)
        tr[...] = to_row(tcol)

    for i in range(n_blocks):
        pltpu.make_async_copy(
            zblk, phi_o.at[pl.ds(i * _TC_BLK, _TC_BLK)], sem_z
        ).wait()
    s = s_ref[0]
    pltpu.make_async_copy(rb0, phi_o.at[pl.ds(s, 1)], sem_r).start()
    pltpu.make_async_copy(rb0, phi_o.at[pl.ds(s, 1)], sem_r).wait()


_row3 = jax.ShapeDtypeStruct((1, _N_WIDTH, _N_NODES), jnp.float32)

_tc_main_call = pl.pallas_call(
    _tc_main_body,
    in_specs=[
        pl.BlockSpec(memory_space=pltpu.SMEM),
        pl.BlockSpec((1, _N_WIDTH), lambda: (0, 0)),
        pl.BlockSpec((_N_WIDTH, _N_NODES), lambda: (0, 0)),
        pl.BlockSpec((16, _NB), lambda: (0, 0)),
    ],
    out_specs=[
        pl.BlockSpec((1, _N_WIDTH), lambda: (0, 0)),
        pl.BlockSpec((1, _N_WIDTH), lambda: (0, 0)),
        pl.BlockSpec((1, _N_WIDTH), lambda: (0, 0)),
        pl.BlockSpec(memory_space=pltpu.MemorySpace.HBM),
        pl.BlockSpec((1, _N_WIDTH, _N_NODES), lambda: (0, 0, 0)),
        pl.BlockSpec((1, _N_WIDTH, _N_NODES), lambda: (0, 0, 0)),
    ],
    out_shape=[_vec, _vec, _vec, _big, _row3, _row3],
    scratch_shapes=[
        pltpu.VMEM((_TC_BLK, _N_WIDTH, _N_NODES), jnp.float32),
        pltpu.VMEM((1, _N_WIDTH, _N_NODES), jnp.float32),
        pltpu.SemaphoreType.DMA,
        pltpu.SemaphoreType.DMA,
    ],
)


def _insert2_body(s_ref, rb1, rb2, dphi_in, ddphi_in, dphi_o, ddphi_o, sem_r):
    del dphi_in, ddphi_in
    s = s_ref[0]
    pltpu.make_async_copy(rb1, dphi_o.at[pl.ds(s, 1)], sem_r).start()
    pltpu.make_async_copy(rb2, ddphi_o.at[pl.ds(s, 1)], sem_r).start()
    pltpu.make_async_copy(rb1, dphi_o.at[pl.ds(s, 1)], sem_r).wait()
    pltpu.make_async_copy(rb2, ddphi_o.at[pl.ds(s, 1)], sem_r).wait()


_insert2_call = pl.pallas_call(
    _insert2_body,
    in_specs=[
        pl.BlockSpec(memory_space=pltpu.SMEM),
        pl.BlockSpec((1, _N_WIDTH, _N_NODES), lambda: (0, 0, 0)),
        pl.BlockSpec((1, _N_WIDTH, _N_NODES), lambda: (0, 0, 0)),
        pl.BlockSpec(memory_space=pltpu.MemorySpace.HBM),
        pl.BlockSpec(memory_space=pltpu.MemorySpace.HBM),
    ],
    out_specs=[
        pl.BlockSpec(memory_space=pltpu.MemorySpace.HBM),
        pl.BlockSpec(memory_space=pltpu.MemorySpace.HBM),
    ],
    out_shape=[_big, _big],
    scratch_shapes=[pltpu.SemaphoreType.DMA],
    input_output_aliases={3: 0, 4: 1},
)


def kernel(x, epoch, sample, weight):
    del epoch
    s = jnp.asarray(sample, jnp.int32).reshape((1,))
    zdphi, zddphi, _ = _sc_fill_call()
    t, dt, ddt, phi_buf, rb1, rb2 = _tc_main_call(
        s, x, weight, jnp.asarray(_INV_TABLE)
    )
    dphi_buf, ddphi_buf = _insert2_call(s, rb1, rb2, zdphi, zddphi)
    return (t, dt, ddt, phi_buf, dphi_buf, ddphi_buf, jnp.float32(_DELTA_X))


# speedup vs baseline: 1.0377x; 1.0109x over previous
import functools

import numpy as np
import jax
import jax.numpy as jnp
from jax import lax
from jax.experimental import pallas as pl
from jax.experimental.pallas import tpu as pltpu
from jax.experimental.pallas import tpu_sc as plsc

_N_WIDTH = 256
_N_ORDER = 8
_N_ELEMENTS = 64
_N_NODES = _N_ELEMENTS * _N_ORDER + 1
_N_COLL = 100
_DELTA_X = 0.5 * _N_ORDER / (_N_NODES - 1)
_NODES = np.linspace(-1.0, 1.0, _N_ORDER + 1)
_NB = _N_ORDER + 1

_N_WORKERS = 32
_HALF = 128
_QUART = 64
_N_ITEMS = 2 * _N_COLL * 2
_ITEMS_PER_W = 13
_TC_BLK = 10


def _inv_table():
    v = np.zeros((16, _NB), np.float32)
    for j in range(_NB):
        for m in range(_NB):
            if j != m:
                v[j, m] = 1.0 / (_NODES[j] - _NODES[m])
    return v


_INV_TABLE = _inv_table()


def _omit_one_products(fs):
    n = len(fs)
    pre = [None] * n
    suf = [None] * n
    acc = None
    for i in range(n):
        pre[i] = acc
        acc = fs[i] if acc is None else acc * fs[i]
    acc = None
    for i in range(n - 1, -1, -1):
        suf[i] = acc
        acc = fs[i] if acc is None else acc * fs[i]
    out = []
    for i in range(n):
        if pre[i] is None:
            out.append(suf[i])
        elif suf[i] is None:
            out.append(pre[i])
        else:
            out.append(pre[i] * suf[i])
    return out


def _basis_rows(x_t, inv_tab):
    j2 = jax.lax.broadcasted_iota(jnp.int32, (16, _N_WIDTH), 0)
    xb = jnp.broadcast_to(x_t, (16, _N_WIDTH)).astype(jnp.float32)
    f = []
    cs = [inv_tab[:, m : m + 1] for m in range(_NB)]
    for m in range(_NB):
        f.append(jnp.where(j2 == m, 1.0, (xb - np.float32(_NODES[m])) * cs[m]))

    phi = f[0]
    for m in range(1, _NB):
        phi = phi * f[m]

    p1 = _omit_one_products(f)
    dphi = cs[0] * p1[0]
    for i in range(1, _NB):
        dphi = dphi + cs[i] * p1[i]

    ddphi = None
    for i in range(_NB):
        idxs = [m for m in range(_NB) if m != i]
        q = _omit_one_products([f[m] for m in idxs])
        inner = None
        for pos, m in enumerate(idxs):
            term = cs[m] * q[pos]
            inner = term if inner is None else inner + term
        term = cs[i] * inner
        ddphi = term if ddphi is None else ddphi + term

    dphi = dphi * np.float32(1.0 / _DELTA_X)
    ddphi = ddphi * np.float32(1.0 / (_DELTA_X * _DELTA_X))
    return phi, dphi, ddphi


def _rows_body(x_ref, w_ref, inv_ref, rb0_o, rb1_o, rb2_o, t_r, dt_r, ddt_r):
    xs = x_ref[...]
    x_shift = (_N_NODES - 1.0) * xs
    id_el = jnp.clip(jnp.floor(x_shift / _N_ORDER), 0.0, _N_ELEMENTS - 1.0)
    nl_f = id_el * _N_ORDER
    x_t = (x_shift - nl_f - 0.5 * _N_ORDER) / (0.5 * _N_ORDER)
    phi, dphi, ddphi = _basis_rows(x_t, inv_ref[...])

    ident = (
        jax.lax.broadcasted_iota(jnp.int32, (_N_WIDTH, _N_WIDTH), 0)
        == jax.lax.broadcasted_iota(jnp.int32, (_N_WIDTH, _N_WIDTH), 1)
    ).astype(jnp.float32)

    def to_col(row):
        return jnp.sum(ident * row, axis=1, keepdims=True)

    def to_row(col):
        return jnp.sum(ident * col, axis=0, keepdims=True)

    nl_col = to_col(nl_f).astype(jnp.int32)
    rel = jax.lax.broadcasted_iota(jnp.int32, (_N_WIDTH, _N_NODES), 1) - nl_col

    bases = (phi, dphi, ddphi)
    dense = [jnp.zeros((_N_WIDTH, _N_NODES), jnp.float32) for _ in range(3)]
    for j in range(_NB):
        mask = rel == j
        for b in range(3):
            colv = to_col(bases[b][j : j + 1, :])
            dense[b] = jnp.where(mask, colv, dense[b])

    rb0_o[...] = dense[0][None]
    rb1_o[...] = dense[1][None]
    rb2_o[...] = dense[2][None]

    w = w_ref[...]
    for d, tr in zip(dense, (t_r, dt_r, ddt_r)):
        tcol = jnp.sum(w * d, axis=1, keepdims=True)
        tr[...] = to_row(tcol)


_row3 = jax.ShapeDtypeStruct((1, _N_WIDTH, _N_NODES), jnp.float32)
_vec = jax.ShapeDtypeStruct((1, _N_WIDTH), jnp.float32)
_big = jax.ShapeDtypeStruct((_N_COLL, _N_WIDTH, _N_NODES), jnp.float32)

_rows_call = pl.pallas_call(
    _rows_body,
    in_specs=[
        pl.BlockSpec((1, _N_WIDTH), lambda: (0, 0)),
        pl.BlockSpec((_N_WIDTH, _N_NODES), lambda: (0, 0)),
        pl.BlockSpec((16, _NB), lambda: (0, 0)),
    ],
    out_specs=[
        pl.BlockSpec((1, _N_WIDTH, _N_NODES), lambda: (0, 0, 0)),
        pl.BlockSpec((1, _N_WIDTH, _N_NODES), lambda: (0, 0, 0)),
        pl.BlockSpec((1, _N_WIDTH, _N_NODES), lambda: (0, 0, 0)),
        pl.BlockSpec((1, _N_WIDTH), lambda: (0, 0)),
        pl.BlockSpec((1, _N_WIDTH), lambda: (0, 0)),
        pl.BlockSpec((1, _N_WIDTH), lambda: (0, 0)),
    ],
    out_shape=[_row3, _row3, _row3, _vec, _vec, _vec],
)


def _sc_fill_body(samp_hbm, rb1_hbm, rb2_hbm, dphi_o, ddphi_o, dummy_o,
                  zbuf, qbuf, svm, sem, sem_q):
    wid = lax.axis_index("s") * 2 + lax.axis_index("c")

    def _zrow(r, carry):
        for c in range(32):
            zbuf[0, r, pl.ds(c * 16, 16)] = jnp.zeros((16,), jnp.float32)
        zbuf[0, r, pl.ds(_N_NODES - 16, 16)] = jnp.zeros((16,), jnp.float32)
        return carry

    lax.fori_loop(0, _HALF, _zrow, 0)

    pltpu.sync_copy(samp_hbm, svm)
    s = svm[...][0]

    outs = (dphi_o, ddphi_o)
    for n in range(_ITEMS_PER_W):
        h = wid * _ITEMS_PER_W + n
        b = h // (2 * _N_COLL)
        rem = h % (2 * _N_COLL)
        row = rem // 2
        half = rem % 2
        real = (h < _N_ITEMS) & (row != s)
        for b_id in range(2):
            dst = outs[b_id].at[
                pl.ds(row, 1), pl.ds(half * _HALF, _HALF), pl.ds(0, _N_NODES)
            ]

            @pl.when(real & (b == b_id))
            def _(dst=dst):
                pltpu.make_async_copy(zbuf, dst, sem).start()

        @pl.when((h >= _N_ITEMS) | (row == s))
        def _():
            pltpu.make_async_copy(zbuf, dummy_o, sem).start()

    for t_id in range(2):
        rb = (rb1_hbm, rb2_hbm)[t_id]
        out = outs[t_id]
        for q in range(4):
            @pl.when(wid == t_id * 4 + q)
            def _(rb=rb, out=out, q=q):
                src = rb.at[
                    pl.ds(0, 1), pl.ds(q * _QUART, _QUART), pl.ds(0, _N_NODES)
                ]
                pltpu.sync_copy(src, qbuf)
                dst = out.at[
                    pl.ds(s, 1), pl.ds(q * _QUART, _QUART), pl.ds(0, _N_NODES)
                ]
                cp = pltpu.make_async_copy(qbuf, dst, sem_q)
                cp.start()
                cp.wait()

    for n in range(_ITEMS_PER_W):
        pltpu.make_async_copy(zbuf, dummy_o, sem).wait()


_sc_fill_call = functools.partial(
    pl.kernel,
    mesh=plsc.VectorSubcoreMesh(
        core_axis_name="c", subcore_axis_name="s", num_cores=2, num_subcores=16
    ),
    out_type=[
        _big,
        _big,
        jax.ShapeDtypeStruct((1, _HALF, _N_NODES), jnp.float32),
    ],
    scratch_types=[
        pltpu.VMEM((1, _HALF, _N_NODES), jnp.float32),
        pltpu.VMEM((1, _QUART, _N_NODES), jnp.float32),
        pltpu.VMEM((16,), jnp.int32),
        pltpu.SemaphoreType.DMA,
        pltpu.SemaphoreType.DMA,
    ],
)(_sc_fill_body)


def _tc_phi_body(s_ref, rb0, phi_o, zblk, sem_z, sem_r):
    zblk[...] = jnp.zeros((_TC_BLK, _N_WIDTH, _N_NODES), jnp.float32)
    n_blocks = _N_COLL // _TC_BLK
    for i in range(n_blocks):
        pltpu.make_async_copy(
            zblk, phi_o.at[pl.ds(i * _TC_BLK, _TC_BLK)], sem_z
        ).start()
    for i in range(n_blocks):
        pltpu.make_async_copy(
            zblk, phi_o.at[pl.ds(i * _TC_BLK, _TC_BLK)], sem_z
        ).wait()
    s = s_ref[0]
    pltpu.make_async_copy(rb0, phi_o.at[pl.ds(s, 1)], sem_r).start()
    pltpu.make_async_copy(rb0, phi_o.at[pl.ds(s, 1)], sem_r).wait()


_tc_phi_call = pl.pallas_call(
    _tc_phi_body,
    in_specs=[
        pl.BlockSpec(memory_space=pltpu.SMEM),
        pl.BlockSpec((1, _N_WIDTH, _N_NODES), lambda: (0, 0, 0)),
    ],
    out_specs=[pl.BlockSpec(memory_space=pltpu.MemorySpace.HBM)],
    out_shape=[_big],
    scratch_shapes=[
        pltpu.VMEM((_TC_BLK, _N_WIDTH, _N_NODES), jnp.float32),
        pltpu.SemaphoreType.DMA,
        pltpu.SemaphoreType.DMA,
    ],
)


def kernel(x, epoch, sample, weight):
    del epoch
    s = jnp.asarray(sample, jnp.int32).reshape((1,))
    samp = jnp.full((16,), sample, jnp.int32)
    rb0, rb1, rb2, t, dt, ddt = _rows_call(x, weight, jnp.asarray(_INV_TABLE))
    dphi_buf, ddphi_buf, _ = _sc_fill_call(samp, rb1, rb2)
    (phi_buf,) = _tc_phi_call(s, rb0)
    return (t, dt, ddt, phi_buf, dphi_buf, ddphi_buf, jnp.float32(_DELTA_X))


# speedup vs baseline: 2.5320x; 2.4400x over previous
import functools

import numpy as np
import jax
import jax.numpy as jnp
from jax import lax
from jax.experimental import pallas as pl
from jax.experimental.pallas import tpu as pltpu
from jax.experimental.pallas import tpu_sc as plsc

_N_WIDTH = 256
_N_ORDER = 8
_N_ELEMENTS = 64
_N_NODES = _N_ELEMENTS * _N_ORDER + 1
_N_COLL = 100
_DELTA_X = 0.5 * _N_ORDER / (_N_NODES - 1)
_NODES = np.linspace(-1.0, 1.0, _N_ORDER + 1)
_NB = _N_ORDER + 1

_N_ITEMS = 2 * _N_COLL
_ITEMS_PER_W = 7
_TC_BLK = 10


def _inv_table():
    v = np.zeros((16, _NB), np.float32)
    for j in range(_NB):
        for m in range(_NB):
            if j != m:
                v[j, m] = 1.0 / (_NODES[j] - _NODES[m])
    return v


_INV_TABLE = _inv_table()


def _omit_one_products(fs):
    n = len(fs)
    pre = [None] * n
    suf = [None] * n
    acc = None
    for i in range(n):
        pre[i] = acc
        acc = fs[i] if acc is None else acc * fs[i]
    acc = None
    for i in range(n - 1, -1, -1):
        suf[i] = acc
        acc = fs[i] if acc is None else acc * fs[i]
    out = []
    for i in range(n):
        if pre[i] is None:
            out.append(suf[i])
        elif suf[i] is None:
            out.append(pre[i])
        else:
            out.append(pre[i] * suf[i])
    return out


def _basis_rows(x_t, inv_tab):
    j2 = jax.lax.broadcasted_iota(jnp.int32, (16, _N_WIDTH), 0)
    xb = jnp.broadcast_to(x_t, (16, _N_WIDTH)).astype(jnp.float32)
    f = []
    cs = [inv_tab[:, m : m + 1] for m in range(_NB)]
    for m in range(_NB):
        f.append(jnp.where(j2 == m, 1.0, (xb - np.float32(_NODES[m])) * cs[m]))

    phi = f[0]
    for m in range(1, _NB):
        phi = phi * f[m]

    p1 = _omit_one_products(f)
    dphi = cs[0] * p1[0]
    for i in range(1, _NB):
        dphi = dphi + cs[i] * p1[i]

    ddphi = None
    for i in range(_NB):
        idxs = [m for m in range(_NB) if m != i]
        q = _omit_one_products([f[m] for m in idxs])
        inner = None
        for pos, m in enumerate(idxs):
            term = cs[m] * q[pos]
            inner = term if inner is None else inner + term
        term = cs[i] * inner
        ddphi = term if ddphi is None else ddphi + term

    dphi = dphi * np.float32(1.0 / _DELTA_X)
    ddphi = ddphi * np.float32(1.0 / (_DELTA_X * _DELTA_X))
    return phi, dphi, ddphi


def _rows_body(x_ref, wt_ref, inv_ref, rb0_o, rb1_o, rb2_o, t_r, dt_r, ddt_r):
    xs = x_ref[...]
    x_shift = (_N_NODES - 1.0) * xs
    id_el = jnp.clip(jnp.floor(x_shift / _N_ORDER), 0.0, _N_ELEMENTS - 1.0)
    nl_f = id_el * _N_ORDER
    x_t = (x_shift - nl_f - 0.5 * _N_ORDER) / (0.5 * _N_ORDER)
    phi, dphi, ddphi = _basis_rows(x_t, inv_ref[...])

    nl_row = nl_f.astype(jnp.int32)
    rel = (
        jax.lax.broadcasted_iota(jnp.int32, (_N_NODES, _N_WIDTH), 0) - nl_row
    )

    bases = (phi, dphi, ddphi)
    dense = [jnp.zeros((_N_NODES, _N_WIDTH), jnp.float32) for _ in range(3)]
    for j in range(_NB):
        mask = rel == j
        for b in range(3):
            dense[b] = jnp.where(mask, bases[b][j : j + 1, :], dense[b])

    rb0_o[...] = dense[0][None]
    rb1_o[...] = dense[1][None]
    rb2_o[...] = dense[2][None]

    wt = wt_ref[...]
    for d, tr in zip(dense, (t_r, dt_r, ddt_r)):
        tr[...] = jnp.sum(wt * d, axis=0, keepdims=True)


_rowT = jax.ShapeDtypeStruct((1, _N_NODES, _N_WIDTH), jnp.float32)
_vec = jax.ShapeDtypeStruct((1, _N_WIDTH), jnp.float32)
_bigT = jax.ShapeDtypeStruct((_N_COLL, _N_NODES, _N_WIDTH), jnp.float32)

_rows_call = pl.pallas_call(
    _rows_body,
    in_specs=[
        pl.BlockSpec((1, _N_WIDTH), lambda: (0, 0)),
        pl.BlockSpec((_N_NODES, _N_WIDTH), lambda: (0, 0)),
        pl.BlockSpec((16, _NB), lambda: (0, 0)),
    ],
    out_specs=[
        pl.BlockSpec((1, _N_NODES, _N_WIDTH), lambda: (0, 0, 0)),
        pl.BlockSpec((1, _N_NODES, _N_WIDTH), lambda: (0, 0, 0)),
        pl.BlockSpec((1, _N_NODES, _N_WIDTH), lambda: (0, 0, 0)),
        pl.BlockSpec((1, _N_WIDTH), lambda: (0, 0)),
        pl.BlockSpec((1, _N_WIDTH), lambda: (0, 0)),
        pl.BlockSpec((1, _N_WIDTH), lambda: (0, 0)),
    ],
    out_shape=[_rowT, _rowT, _rowT, _vec, _vec, _vec],
)


def _sc_fill_body(samp_hbm, zeros_hbm, rb1_hbm, rb2_hbm,
                  dphi_o, ddphi_o, dummy_o, zsh, rsh, svm, sem, sem_q):
    wid = lax.axis_index("s") * 2 + lax.axis_index("c")
    sid = lax.axis_index("s")

    @pl.when(sid == 0)
    def _():
        pltpu.sync_copy(zeros_hbm, zsh)

    plsc.subcore_barrier()

    pltpu.sync_copy(samp_hbm, svm)
    s = svm[...][0]

    outs = (dphi_o, ddphi_o)
    for n in range(_ITEMS_PER_W):
        h = wid * _ITEMS_PER_W + n
        b = h // _N_COLL
        face = h % _N_COLL
        real = (h < _N_ITEMS) & (face != s)
        for b_id in range(2):
            dst = outs[b_id].at[pl.ds(face, 1)]

            @pl.when(real & (b == b_id))
            def _(dst=dst):
                pltpu.make_async_copy(zsh, dst, sem).start()

        @pl.when((h >= _N_ITEMS) | (face == s))
        def _():
            pltpu.make_async_copy(zsh, dummy_o, sem).start()

    for t_id in range(2):
        rb = (rb1_hbm, rb2_hbm)[t_id]
        out = outs[t_id]

        @pl.when(wid == t_id)
        def _(rb=rb, out=out):
            pltpu.sync_copy(rb, rsh)
            cp = pltpu.make_async_copy(rsh, out.at[pl.ds(s, 1)], sem_q)
            cp.start()
            cp.wait()

    for n in range(_ITEMS_PER_W):
        pltpu.make_async_copy(zsh, dummy_o, sem).wait()


_sc_fill_call = functools.partial(
    pl.kernel,
    mesh=plsc.VectorSubcoreMesh(
        core_axis_name="c", subcore_axis_name="s", num_cores=2, num_subcores=16
    ),
    out_type=[
        _bigT,
        _bigT,
        jax.ShapeDtypeStruct((1, _N_NODES, _N_WIDTH), jnp.float32),
    ],
    scratch_types=[
        pltpu.VMEM_SHARED((1, _N_NODES, _N_WIDTH), jnp.float32),
        pltpu.VMEM_SHARED((1, _N_NODES, _N_WIDTH), jnp.float32),
        pltpu.VMEM((16,), jnp.int32),
        pltpu.SemaphoreType.DMA,
        pltpu.SemaphoreType.DMA,
    ],
)(_sc_fill_body)


def _tc_phi_body(s_ref, rb0_ref, phi_o):
    i = pl.program_id(0)
    s = s_ref[0]
    base = i * _TC_BLK
    phi_o[...] = jnp.zeros((_TC_BLK, _N_NODES, _N_WIDTH), jnp.float32)

    @pl.when((s >= base) & (s < base + _TC_BLK))
    def _():
        phi_o[pl.ds(s - base, 1)] = rb0_ref[...]


_tc_phi_call = pl.pallas_call(
    _tc_phi_body,
    grid=(_N_COLL // _TC_BLK,),
    in_specs=[
        pl.BlockSpec(memory_space=pltpu.SMEM),
        pl.BlockSpec((1, _N_NODES, _N_WIDTH), lambda i: (0, 0, 0)),
    ],
    out_specs=[
        pl.BlockSpec((_TC_BLK, _N_NODES, _N_WIDTH), lambda i: (i, 0, 0)),
    ],
    out_shape=[_bigT],
)


def kernel(x, epoch, sample, weight):
    del epoch
    s = jnp.asarray(sample, jnp.int32).reshape((1,))
    samp = jnp.full((16,), sample, jnp.int32)
    wt = jnp.transpose(weight)
    zrow = jnp.zeros((1, _N_NODES, _N_WIDTH), jnp.float32)
    rb0, rb1, rb2, t, dt, ddt = _rows_call(x, wt, jnp.asarray(_INV_TABLE))
    dphi_t, ddphi_t, _ = _sc_fill_call(samp, zrow, rb1, rb2)
    (phi_t,) = _tc_phi_call(s, rb0)
    phi_buf = jnp.transpose(phi_t, (0, 2, 1))
    dphi_buf = jnp.transpose(dphi_t, (0, 2, 1))
    ddphi_buf = jnp.transpose(ddphi_t, (0, 2, 1))
    return (t, dt, ddt, phi_buf, dphi_buf, ddphi_buf, jnp.float32(_DELTA_X))
